# Initial kernel scaffold; baseline (speedup 1.0000x reference)
import jax
import jax.numpy as jnp
from jax.experimental import pallas as pl


def kernel(x, edge_index, W1, b1, W2, b2):
    raise NotImplementedError("write your pallas kernel here")


# speedup vs baseline: 20.8621x; 20.8621x over previous
import functools

import jax
import jax.numpy as jnp
from jax import lax
from jax.experimental import pallas as pl
from jax.experimental.pallas import tpu as pltpu
from jax.experimental.pallas import tpu_sc as plsc

N = 10000
E = 320000
NC = 2
NS = 16
NW = NC * NS
EPW = E // NW
CH = 125
NCH = EPW // CH
RA = 624
LAST_BASE = RA * (NS - 1)
LAST_ROWS = N - LAST_BASE


def _tilewise_copy(s, src_at, dst_at):
    @pl.when(s < NS - 1)
    def _():
        pltpu.sync_copy(src_at(s * RA, RA), dst_at(s * RA, RA))

    @pl.when(s == NS - 1)
    def _():
        pltpu.sync_copy(src_at(LAST_BASE, LAST_ROWS), dst_at(LAST_BASE, LAST_ROWS))

_MESH = plsc.VectorSubcoreMesh(
    core_axis_name="c", subcore_axis_name="s", num_cores=NC, num_subcores=NS
)


@functools.partial(
    pl.kernel,
    out_type=jax.ShapeDtypeStruct((NC, N, 16), jnp.float32),
    mesh=_MESH,
    scratch_types=[
        pltpu.VMEM((NCH, CH), jnp.int32),
        pltpu.VMEM((CH, 16), jnp.float32),
        pltpu.VMEM_SHARED((N, 16), jnp.float32),
    ],
)
def _deg_kernel(dst_hbm, ones_hbm, zeros_hbm, out_hbm, dst_v, ones_v, acc):
    c = lax.axis_index("c")
    s = lax.axis_index("s")
    wid = s * NC + c
    _tilewise_copy(s, lambda b, r: zeros_hbm.at[pl.ds(b, r)],
                   lambda b, r: acc.at[pl.ds(b, r)])
    pltpu.sync_copy(dst_hbm.at[wid], dst_v)
    pltpu.sync_copy(ones_hbm, ones_v)
    plsc.subcore_barrier()

    def step(j, carry):
        pltpu.sync_copy(ones_v, acc.at[dst_v.at[j]], add=True)
        return carry

    lax.fori_loop(0, NCH, step, 0)
    plsc.subcore_barrier()
    _tilewise_copy(s, lambda b, r: acc.at[pl.ds(b, r)],
                   lambda b, r: out_hbm.at[c, pl.ds(b, r)])


def _make_aggregate(F):
    @functools.partial(
        pl.kernel,
        out_type=jax.ShapeDtypeStruct((NC, N, F), jnp.float32),
        mesh=_MESH,
        scratch_types=[
            pltpu.VMEM((NCH, CH), jnp.int32),
            pltpu.VMEM((NCH, CH), jnp.int32),
            pltpu.VMEM((CH, F), jnp.float32),
            pltpu.VMEM_SHARED((N, F), jnp.float32),
            pltpu.SemaphoreType.DMA,
        ],
    )
    def agg(table_hbm, src_hbm, dst_hbm, zeros_hbm, out_hbm,
            src_v, dst_v, rows, acc, sem):
        c = lax.axis_index("c")
        s = lax.axis_index("s")
        wid = s * NC + c
        _tilewise_copy(s, lambda b, r: zeros_hbm.at[pl.ds(b, r)],
                       lambda b, r: acc.at[pl.ds(b, r)])
        pltpu.sync_copy(src_hbm.at[wid], src_v)
        pltpu.sync_copy(dst_hbm.at[wid], dst_v)
        plsc.subcore_barrier()

        def step(j, carry):
            pltpu.async_copy(table_hbm.at[src_v.at[j]], rows, sem).wait()
            pltpu.sync_copy(rows, acc.at[dst_v.at[j]], add=True)
            return carry

        lax.fori_loop(0, NCH, step, 0)
        plsc.subcore_barrier()
        _tilewise_copy(s, lambda b, r: acc.at[pl.ds(b, r)],
                       lambda b, r: out_hbm.at[c, pl.ds(b, r)])

    return agg


_agg128 = _make_aggregate(128)

_B = 2000


def _mm_scale_body(x_ref, w_ref, d0_ref, d1_ref, o_ref):
    dinv = lax.rsqrt(d0_ref[...][:, :1] + d1_ref[...][:, :1] + 1.0)
    o_ref[...] = (
        jnp.dot(x_ref[...], w_ref[...], preferred_element_type=jnp.float32) * dinv
    )


def _mm_scale(x, w, d0, d1):
    K = x.shape[1]
    M = w.shape[1]
    return pl.pallas_call(
        _mm_scale_body,
        grid=(N // _B,),
        in_specs=[
            pl.BlockSpec((_B, K), lambda i: (i, 0)),
            pl.BlockSpec((K, M), lambda i: (0, 0)),
            pl.BlockSpec((_B, 16), lambda i: (i, 0)),
            pl.BlockSpec((_B, 16), lambda i: (i, 0)),
        ],
        out_specs=pl.BlockSpec((_B, M), lambda i: (i, 0)),
        out_shape=jax.ShapeDtypeStruct((N, M), jnp.float32),
    )(x, w, d0, d1)


def _layer_mid_body(p0_ref, p1_ref, hs_ref, d0_ref, d1_ref, b_ref, w_ref, o_ref):
    dinv = lax.rsqrt(d0_ref[...][:, :1] + d1_ref[...][:, :1] + 1.0)
    h = jnp.maximum(
        (p0_ref[...] + p1_ref[...] + hs_ref[...]) * dinv + b_ref[...], 0.0
    )
    o_ref[...] = (
        jnp.dot(h, w_ref[...], preferred_element_type=jnp.float32) * dinv
    )


def _layer_mid(p0, p1, hs, d0, d1, b, w):
    K = hs.shape[1]
    M = w.shape[1]
    return pl.pallas_call(
        _layer_mid_body,
        grid=(N // _B,),
        in_specs=[
            pl.BlockSpec((_B, K), lambda i: (i, 0)),
            pl.BlockSpec((_B, K), lambda i: (i, 0)),
            pl.BlockSpec((_B, K), lambda i: (i, 0)),
            pl.BlockSpec((_B, 16), lambda i: (i, 0)),
            pl.BlockSpec((_B, 16), lambda i: (i, 0)),
            pl.BlockSpec((1, K), lambda i: (0, 0)),
            pl.BlockSpec((K, M), lambda i: (0, 0)),
        ],
        out_specs=pl.BlockSpec((_B, M), lambda i: (i, 0)),
        out_shape=jax.ShapeDtypeStruct((N, M), jnp.float32),
    )(p0, p1, hs, d0, d1, b, w)


def _final_body(p0_ref, p1_ref, hs_ref, d0_ref, d1_ref, b_ref, o_ref):
    dinv = lax.rsqrt(d0_ref[...][:, :1] + d1_ref[...][:, :1] + 1.0)
    M = o_ref.shape[1]
    s = (p0_ref[...][:, :M] + p1_ref[...][:, :M] + hs_ref[...][:, :M])
    o_ref[...] = s * dinv + b_ref[...]


def _final(p0, p1, hs, d0, d1, b):
    M = b.shape[1]
    K = hs.shape[1]
    return pl.pallas_call(
        _final_body,
        grid=(N // _B,),
        in_specs=[
            pl.BlockSpec((_B, K), lambda i: (i, 0)),
            pl.BlockSpec((_B, K), lambda i: (i, 0)),
            pl.BlockSpec((_B, K), lambda i: (i, 0)),
            pl.BlockSpec((_B, 16), lambda i: (i, 0)),
            pl.BlockSpec((_B, 16), lambda i: (i, 0)),
            pl.BlockSpec((1, M), lambda i: (0, 0)),
        ],
        out_specs=pl.BlockSpec((_B, M), lambda i: (i, 0)),
        out_shape=jax.ShapeDtypeStruct((N, M), jnp.float32),
    )(p0, p1, hs, d0, d1, b)


def kernel(x, edge_index, W1, b1, W2, b2):
    ei = edge_index.astype(jnp.int32)
    src = ei[0].reshape(NW, NCH, CH)
    dst = ei[1].reshape(NW, NCH, CH)

    ones16 = jnp.ones((CH, 16), jnp.float32)
    zeros16 = jnp.zeros((N, 16), jnp.float32)
    zeros128 = jnp.zeros((N, 128), jnp.float32)
    W2p = jnp.pad(W2, ((0, 0), (0, 128 - W2.shape[1])))

    degp = _deg_kernel(dst, ones16, zeros16)
    d0 = degp[0]
    d1 = degp[1]

    hs1 = _mm_scale(x, W1, d0, d1)
    p1 = _agg128(hs1, src, dst, zeros128)
    hs2 = _layer_mid(p1[0], p1[1], hs1, d0, d1, b1.reshape(1, -1), W2p)
    p2 = _agg128(hs2, src, dst, zeros128)
    z = _final(p2[0], p2[1], hs2, d0, d1, b2.reshape(1, -1))
    return z
